# baseline (speedup 1.0000x reference)
import jax
import jax.numpy as jnp
from jax.experimental import pallas as pl
from jax.experimental.pallas import tpu as pltpu

_NCHUNK = 8
_SLOTS = 8


def kernel(inputs, pos_embedding):
    B, S = inputs.shape
    P, D = pos_embedding.shape
    CS = S // _NCHUNK

    def body(table_hbm, out_hbm, buf, insem, outsem):
        def in_copy(j):
            return pltpu.make_async_copy(
                table_hbm.at[pl.ds(j * CS, CS), :],
                buf.at[j % _SLOTS],
                insem.at[j % _SLOTS],
            )

        def out_copies(j):
            return [
                pltpu.make_async_copy(
                    buf.at[j % _SLOTS],
                    out_hbm.at[b, pl.ds(j * CS, CS), :],
                    outsem.at[j % _SLOTS],
                )
                for b in range(B)
            ]

        pending = {}
        in_copy(0).start()
        for j in range(_NCHUNK):
            nxt = j + 1
            if nxt < _NCHUNK:
                prev = nxt - _SLOTS
                if prev >= 0:
                    for c in pending.pop(prev):
                        c.wait()
                in_copy(nxt).start()
            in_copy(j).wait()
            cs = out_copies(j)
            for c in cs:
                c.start()
            pending[j] = cs
        for j in sorted(pending):
            for c in pending[j]:
                c.wait()

    out = pl.pallas_call(
        body,
        in_specs=[pl.BlockSpec(memory_space=pl.ANY)],
        out_specs=pl.BlockSpec(memory_space=pl.ANY),
        out_shape=jax.ShapeDtypeStruct((B, S, D), pos_embedding.dtype),
        scratch_shapes=[
            pltpu.VMEM((_SLOTS, CS, D), pos_embedding.dtype),
            pltpu.SemaphoreType.DMA((_SLOTS,)),
            pltpu.SemaphoreType.DMA((_SLOTS,)),
        ],
    )(pos_embedding)
    return out


# speedup vs baseline: 1.0151x; 1.0151x over previous
import jax
import jax.numpy as jnp
from jax.experimental import pallas as pl
from jax.experimental.pallas import tpu as pltpu

_NCHUNK = 2
_SLOTS = 2


def kernel(inputs, pos_embedding):
    B, S = inputs.shape
    P, D = pos_embedding.shape
    CS = S // _NCHUNK

    def body(table_hbm, out_hbm, buf, insem, outsem):
        def in_copy(j):
            return pltpu.make_async_copy(
                table_hbm.at[pl.ds(j * CS, CS), :],
                buf.at[j % _SLOTS],
                insem.at[j % _SLOTS],
            )

        def out_copies(j):
            return [
                pltpu.make_async_copy(
                    buf.at[j % _SLOTS],
                    out_hbm.at[b, pl.ds(j * CS, CS), :],
                    outsem.at[j % _SLOTS],
                )
                for b in range(B)
            ]

        pending = {}
        in_copy(0).start()
        for j in range(_NCHUNK):
            nxt = j + 1
            if nxt < _NCHUNK:
                prev = nxt - _SLOTS
                if prev >= 0:
                    for c in pending.pop(prev):
                        c.wait()
                in_copy(nxt).start()
            in_copy(j).wait()
            cs = out_copies(j)
            for c in cs:
                c.start()
            pending[j] = cs
        for j in sorted(pending):
            for c in pending[j]:
                c.wait()

    out = pl.pallas_call(
        body,
        in_specs=[pl.BlockSpec(memory_space=pl.ANY)],
        out_specs=pl.BlockSpec(memory_space=pl.ANY),
        out_shape=jax.ShapeDtypeStruct((B, S, D), pos_embedding.dtype),
        scratch_shapes=[
            pltpu.VMEM((_SLOTS, CS, D), pos_embedding.dtype),
            pltpu.SemaphoreType.DMA((_SLOTS,)),
            pltpu.SemaphoreType.DMA((_SLOTS,)),
        ],
    )(pos_embedding)
    return out
